# Initial kernel scaffold; baseline (speedup 1.0000x reference)
import jax
import jax.numpy as jnp
from jax.experimental import pallas as pl


def kernel(x, w):
    raise NotImplementedError("write your pallas kernel here")


# speedup vs baseline: 2.6441x; 2.6441x over previous
import jax
import jax.numpy as jnp
from jax.experimental import pallas as pl


_ROWS = 16384
_COLS = 4096
_K = 128
_BLOCK_R = 1024


def _select_body(x_ref, s_ref, o_ref):
    o_ref[...] = jnp.dot(
        x_ref[...], s_ref[...], preferred_element_type=jnp.float32
    )


def kernel(x, w):
    b, srows, cols = x.shape
    k = w.shape[0]
    xf = x.reshape(b * srows, cols)
    sel = (
        jax.lax.broadcasted_iota(jnp.int32, (cols, k), 0) == w[None, :]
    ).astype(jnp.float32)

    grid = (xf.shape[0] // _BLOCK_R,)
    out = pl.pallas_call(
        _select_body,
        grid=grid,
        in_specs=[
            pl.BlockSpec((_BLOCK_R, cols), lambda i: (i, 0)),
            pl.BlockSpec((cols, k), lambda i: (0, 0)),
        ],
        out_specs=pl.BlockSpec((_BLOCK_R, k), lambda i: (i, 0)),
        out_shape=jax.ShapeDtypeStruct((xf.shape[0], k), jnp.float32),
    )(xf, sel)
    return out.reshape(b, srows, k)
